# baseline (speedup 1.0000x reference)
import functools

import jax
import jax.numpy as jnp
import numpy as np
from jax import lax
from jax.experimental import pallas as pl
from jax.experimental.pallas import tpu as pltpu
from jax.experimental.pallas import tpu_sc as plsc

_GRID = (256, 256, 32)
_FREE_INDEX = 17
_MIN_DISTANCE = 2.5
_N_STEPS = 256
_NVOX = _GRID[0] * _GRID[1] * _GRID[2]
_HALF = _NVOX // 2
_SENT_LIN = _NVOX
_NWORDS = _NVOX // 32
_BM_PAD = _NWORDS + 128

_LO32 = np.array([-51.2, -51.2, -3.0], dtype=np.float32)
_UP32 = np.array([51.2, 51.2, 3.4], dtype=np.float32)
_VS32 = (_UP32 - _LO32) / np.array(_GRID, dtype=np.float32)
_MAX_DIST = float(np.sqrt(np.sum(np.square(_UP32 - _LO32), dtype=np.float32)))

_LOWER = jnp.asarray(_LO32)
_UPPER = jnp.asarray(_UP32)

_N = 16384
_QUART = _NVOX // 4
_RAYS_T = _N // 16
_CH_RAYS = 16
_CH_WORDS = _CH_RAYS * _N_STEPS


def _pack_body(occ_ref, out_ref):
    bits = (occ_ref[...] != _FREE_INDEX).astype(jnp.int32)
    z = lax.broadcasted_iota(jnp.int32, bits.shape, 0)
    out_ref[...] = jnp.sum(bits << z, axis=0, keepdims=True)[None]


def _pack_bitmask(occ_t):
    words = pl.pallas_call(
        _pack_body,
        grid=(64,),
        in_specs=[pl.BlockSpec((32, 1024), lambda i: (0, i))],
        out_specs=pl.BlockSpec((1, 1, 1024), lambda i: (i, 0, 0)),
        out_shape=jax.ShapeDtypeStruct((64, 1, 1024), jnp.int32),
    )(occ_t)
    return jnp.concatenate(
        [words.reshape(_NWORDS), jnp.zeros((_BM_PAD - _NWORDS,), jnp.int32)])


def _codes_body(sx, sy, sz, dx, dy, dz, rl, ts_ref, out_ref, smax_ref):
    t = ts_ref[...]
    qx = (sx[...] + dx[...] * t - _LO32[0].item()) / _VS32[0].item()
    qy = (sy[...] + dy[...] * t - _LO32[1].item()) / _VS32[1].item()
    qz = (sz[...] + dz[...] * t - _LO32[2].item()) / _VS32[2].item()
    ix = jnp.floor(qx).astype(jnp.int32)
    iy = jnp.floor(qy).astype(jnp.int32)
    iz = jnp.floor(qz).astype(jnp.int32)
    inb = ((ix >= 0) & (ix < _GRID[0]) & (iy >= 0) & (iy < _GRID[1])
           & (iz >= 0) & (iz < _GRID[2]))
    lin = (ix * _GRID[1] + iy) * _GRID[2] + iz
    lin_s = jnp.where(inb, lin, _SENT_LIN)
    tlt = (t < rl[...]).astype(jnp.int32)
    out_ref[...] = (lin_s << 1) | tlt
    smax_ref[...] = jnp.sum(tlt, axis=1, keepdims=True)


def _make_codes(sx, sy, sz, dx, dy, dz, rlen, ts):
    per_ray = pl.BlockSpec((256, 1), lambda i: (i, 0))
    return pl.pallas_call(
        _codes_body,
        grid=(64,),
        in_specs=[per_ray] * 7 + [pl.BlockSpec((1, 256), lambda i: (0, 0))],
        out_specs=(pl.BlockSpec((256, 256), lambda i: (i, 0)), per_ray),
        out_shape=(jax.ShapeDtypeStruct((_N, _N_STEPS), jnp.int32),
                   jax.ShapeDtypeStruct((_N, 1), jnp.int32)),
    )(sx, sy, sz, dx, dy, dz, rlen, ts)


def _sc_body(codes_hbm, smax_hbm, bm_hbm, first_hbm, ncnt_hbm, grid_hbm,
             bm_v, code_a, code_b, idx_v, val_v, zbuf, first_v, ncnt_v,
             smax_v, grid_s, sem_a, sem_b, scat_sem):
    cc = lax.axis_index("c")
    s = lax.axis_index("s")
    lanes = lax.iota(jnp.int32, 16)
    l256 = lanes * _N_STEPS

    def _z(i, _):
        zbuf[pl.ds(i * 16, 16)] = jnp.zeros((16,), jnp.int32)
        return _
    lax.fori_loop(0, 256, _z, None)

    def _zg(k, _):
        pltpu.sync_copy(zbuf, grid_s.at[pl.ds(s * 32768 + k * 4096, 4096)])
        return _
    lax.fori_loop(0, 8, _zg, None)

    pltpu.sync_copy(bm_hbm, bm_v)
    pltpu.sync_copy(smax_hbm.at[pl.ds(s * _RAYS_T, _RAYS_T)], smax_v)
    plsc.subcore_barrier()

    def occ_at(code_v_, sstep):
        code = plsc.load_gather(code_v_, [l256 + sstep])
        c1 = code >> 1
        occw = plsc.load_gather(bm_v, [c1 >> 5])
        occb = lax.shift_right_logical(occw, c1 & 31) & 1
        return code, c1, occb

    def code_slice(ci):
        base = (s * _RAYS_T + ci * _CH_RAYS) * _N_STEPS
        return codes_hbm.at[pl.ds(base, _CH_WORDS)]

    def chunk_body(ci, code_v, code_sem):
        smax16 = smax_v[pl.ds(ci * _CH_RAYS, 16)]
        ka = (jnp.max(smax16) + 7) >> 3

        nf0 = jnp.ones((16,), jnp.int32)
        acc0 = jnp.zeros((16,), jnp.int32)

        def blockA(b, carry):
            nf, first, acc, woff = carry
            for k in range(8):
                code, c1, occb = occ_at(code_v, b * 8 + k)
                nf = nf & (1 - occb)
                first = first + nf
                bb = occb & code & 1
                acc = acc + bb
                sb = (bb == 1) & ((c1 >> 20) == cc)
                u = c1 & (_HALF - 1)
                sval = jnp.where((u >> 19) == 0, 1, 1 << 16)
                plsc.store_compressed(idx_v.at[pl.ds(woff, 16)],
                                      u & (_QUART - 1), mask=sb)
                plsc.store_compressed(val_v.at[pl.ds(woff, 16)], sval,
                                      mask=sb)
                woff = woff + plsc.all_reduce_population_count(sb)[0]
            return nf, first, acc, woff

        nf, first, acc, woff = lax.fori_loop(
            0, ka, blockA, (nf0, acc0, acc0, jnp.int32(0)))

        def condB(st):
            v, nf, first = st
            return (v < _N_STEPS) & (jnp.max(nf) > 0)

        def bodyB(st):
            v, nf, first = st
            for k in range(8):
                _, _, occb = occ_at(code_v, v + k)
                nf = nf & (1 - occb)
                first = first + nf
            return v + 8, nf, first

        first = lax.while_loop(condB, bodyB, (ka * 8, nf, first))[2]

        first_v[pl.ds(ci * _CH_RAYS, 16)] = first
        ncnt_v[pl.ds(ci * _CH_RAYS, 16)] = acc

        for k in range(8):
            val_v[pl.ds(woff + k * 16, 16)] = jnp.zeros((16,), jnp.int32)
        nr = (woff + 127) >> 7

        def flush(j, _):
            pltpu.sync_copy(val_v.at[pl.ds(j * 128, 128)],
                            grid_s.at[idx_v.at[pl.ds(j * 128, 128)]],
                            add=True)
            return _
        lax.fori_loop(0, nr, flush, None)

    def _zi(j, _):
        for k in range(8):
            idx_v[pl.ds(j * 128 + k * 16, 16)] = jnp.zeros((16,), jnp.int32)
        return _
    lax.fori_loop(0, 32, _zi, None)

    pltpu.async_copy(code_slice(0), code_a, sem_a)

    def pair_body(i, _):
        ci0 = i * 2
        pltpu.make_async_copy(code_slice(ci0), code_a, sem_a).wait()
        pltpu.async_copy(code_slice(ci0 + 1), code_b, sem_b)
        chunk_body(ci0, code_a, sem_a)

        pltpu.make_async_copy(code_slice(ci0 + 1), code_b, sem_b).wait()

        @pl.when(i < _RAYS_T // _CH_RAYS // 2 - 1)
        def _():
            pltpu.async_copy(code_slice(ci0 + 2), code_a, sem_a)
        chunk_body(ci0 + 1, code_b, sem_b)
        return _

    lax.fori_loop(0, _RAYS_T // _CH_RAYS // 2, pair_body, None)

    @pl.when(cc == 0)
    def _():
        pltpu.sync_copy(first_v, first_hbm.at[pl.ds(s * _RAYS_T, _RAYS_T)])
        pltpu.sync_copy(ncnt_v, ncnt_hbm.at[pl.ds(s * _RAYS_T, _RAYS_T)])

    plsc.subcore_barrier()
    pltpu.sync_copy(grid_s.at[pl.ds(s * 32768, 32768)],
                    grid_hbm.at[cc, pl.ds(s * 32768, 32768)])


def _sc_march(codes, smax, bm):
    mesh = plsc.VectorSubcoreMesh(core_axis_name="c", subcore_axis_name="s",
                                  num_cores=2, num_subcores=16)
    fn = pl.kernel(
        _sc_body,
        out_type=(jax.ShapeDtypeStruct((_N,), jnp.int32),
                  jax.ShapeDtypeStruct((_N,), jnp.int32),
                  jax.ShapeDtypeStruct((2, _QUART), jnp.int32)),
        mesh=mesh,
        compiler_params=pltpu.CompilerParams(needs_layout_passes=False),
        scratch_types=[
            pltpu.VMEM((_BM_PAD,), jnp.int32),
            pltpu.VMEM((_CH_WORDS,), jnp.int32),
            pltpu.VMEM((_CH_WORDS,), jnp.int32),
            pltpu.VMEM((4224,), jnp.int32),
            pltpu.VMEM((4224,), jnp.int32),
            pltpu.VMEM((4096,), jnp.int32),
            pltpu.VMEM((_RAYS_T,), jnp.int32),
            pltpu.VMEM((_RAYS_T,), jnp.int32),
            pltpu.VMEM((_RAYS_T,), jnp.int32),
            pltpu.VMEM_SHARED((_QUART,), jnp.int32),
            pltpu.SemaphoreType.DMA,
            pltpu.SemaphoreType.DMA,
            pltpu.SemaphoreType.DMA,
        ],
    )
    return fn(codes.reshape(_N * _N_STEPS), smax.reshape(_N), bm)


def _combine_body(g_ref, l0_ref, h0_ref, l1_ref, h1_ref):
    g = g_ref[...]
    mask = jnp.int32(0xFFFF)
    l0_ref[...] = (g[0] & mask).astype(jnp.float32)
    h0_ref[...] = lax.shift_right_logical(g[0], 16).astype(jnp.float32)
    l1_ref[...] = (g[1] & mask).astype(jnp.float32)
    h1_ref[...] = lax.shift_right_logical(g[1], 16).astype(jnp.float32)


def _combine(grid2):
    g3 = grid2.reshape(2, 4096, 128)
    spec = pl.BlockSpec((256, 128), lambda i: (i, 0))
    sds = jax.ShapeDtypeStruct((4096, 128), jnp.float32)
    parts = pl.pallas_call(
        _combine_body,
        grid=(16,),
        in_specs=[pl.BlockSpec((2, 256, 128), lambda i: (0, i, 0))],
        out_specs=(spec, spec, spec, spec),
        out_shape=(sds, sds, sds, sds),
    )(g3)
    return jnp.concatenate([p.reshape(_QUART) for p in parts])


def _final_body(first_ref, t_box_ref, mask_ref, ld_ref, rend_ref, fin_ref):
    fs = first_ref[...]
    hh = fs < _N_STEPS
    md = jnp.float32(_MAX_DIST)
    th = (fs.astype(jnp.float32) + 0.5) / _N_STEPS * md
    r = jnp.minimum(jnp.where(hh, th, md), md)
    rend_ref[...] = jnp.minimum(r, t_box_ref[...])
    fin_ref[...] = hh & mask_ref[...] & (ld_ref[...] > _MIN_DISTANCE)


def _finalize(first_s, t_box, mask, ld):
    return pl.pallas_call(
        _final_body,
        out_shape=(jax.ShapeDtypeStruct((1, _N), jnp.float32),
                   jax.ShapeDtypeStruct((1, _N), jnp.bool_)),
    )(first_s, t_box, mask, ld)


def _transform(T, pts):
    return jnp.einsum('bij,bnj->bni', T[:, :3, :3], pts) + T[:, None, :3, 3]


def kernel(occupancy, points_lidar, points_mask, ego_from_lidar):
    B, N, _ = points_lidar.shape

    start = _transform(ego_from_lidar, jnp.zeros_like(points_lidar))
    end = _transform(ego_from_lidar, points_lidar)
    ray = end - start
    ray_len = jnp.linalg.norm(ray, axis=-1)
    dirn = ray / jnp.maximum(ray_len[..., None], 1e-8)
    ts = (jnp.arange(_N_STEPS, dtype=jnp.float32) + 0.5) / _N_STEPS * _MAX_DIST

    occ_t = occupancy.reshape(_NWORDS, 32).T
    bm = _pack_bitmask(occ_t)

    col = lambda a: a.reshape(N, 1)
    codes, smax = _make_codes(col(start[0, :, 0]), col(start[0, :, 1]),
                              col(start[0, :, 2]), col(dirn[0, :, 0]),
                              col(dirn[0, :, 1]), col(dirn[0, :, 2]),
                              col(ray_len[0]), ts.reshape(1, _N_STEPS))

    first_s, ncnt, grid2 = _sc_march(codes, smax, bm)

    voxel_trans = _combine(grid2).reshape(B, *_GRID)
    num_transmissions = ncnt.astype(jnp.float32).reshape(B, N)

    safe = jnp.where(jnp.abs(dirn) < 1e-8, 1e-8, dirn)
    t_up = (_UPPER - start) / safe
    t_lo = (_LOWER - start) / safe
    t_exit = jnp.where(dirn > 0, t_up, t_lo)
    t_exit = jnp.where(jnp.abs(dirn) < 1e-8, jnp.full_like(t_exit, 1e9), t_exit)
    t_box = jnp.maximum(jnp.min(t_exit, axis=-1), 0.0)

    lidar_depth = jnp.linalg.norm(points_lidar, axis=-1)
    in_vol = lambda p: jnp.all((p >= _LOWER) & (p <= _UPPER), axis=-1)
    points_in_volume = (in_vol(end) & in_vol(start) & points_mask
                        & (lidar_depth > _MIN_DISTANCE))

    rendered, finite_depth = _finalize(first_s.reshape(B, N), t_box,
                                       points_mask, lidar_depth)

    return (rendered, num_transmissions, lidar_depth, points_in_volume,
            finite_depth, voxel_trans)


# speedup vs baseline: 1.2813x; 1.2813x over previous
import functools

import jax
import jax.numpy as jnp
import numpy as np
from jax import lax
from jax.experimental import pallas as pl
from jax.experimental.pallas import tpu as pltpu
from jax.experimental.pallas import tpu_sc as plsc

_GRID = (256, 256, 32)
_FREE_INDEX = 17
_MIN_DISTANCE = 2.5
_N_STEPS = 256
_NVOX = _GRID[0] * _GRID[1] * _GRID[2]
_HALF = _NVOX // 2
_SENT_LIN = _NVOX
_NWORDS = _NVOX // 32
_BM_PAD = _NWORDS + 128

_LO32 = np.array([-51.2, -51.2, -3.0], dtype=np.float32)
_UP32 = np.array([51.2, 51.2, 3.4], dtype=np.float32)
_VS32 = (_UP32 - _LO32) / np.array(_GRID, dtype=np.float32)
_MAX_DIST = float(np.sqrt(np.sum(np.square(_UP32 - _LO32), dtype=np.float32)))

_LOWER = jnp.asarray(_LO32)
_UPPER = jnp.asarray(_UP32)

_N = 16384
_QUART = _NVOX // 4
_RAYS_T = _N // 16
_CH_RAYS = 16
_CH_WORDS = _CH_RAYS * _N_STEPS


def _pack_body(occ_ref, out_ref):
    bits = (occ_ref[...] != _FREE_INDEX).astype(jnp.int32)
    z = lax.broadcasted_iota(jnp.int32, bits.shape, 0)
    out_ref[...] = jnp.sum(bits << z, axis=0, keepdims=True)[None]


def _pack_bitmask(occ_t):
    words = pl.pallas_call(
        _pack_body,
        grid=(64,),
        in_specs=[pl.BlockSpec((32, 1024), lambda i: (0, i))],
        out_specs=pl.BlockSpec((1, 1, 1024), lambda i: (i, 0, 0)),
        out_shape=jax.ShapeDtypeStruct((64, 1, 1024), jnp.int32),
    )(occ_t)
    return jnp.concatenate(
        [words.reshape(_NWORDS), jnp.zeros((_BM_PAD - _NWORDS,), jnp.int32)])


def _codes_body(sx, sy, sz, dx, dy, dz, rl, ts_ref, out_ref, smax_ref):
    t = ts_ref[...]
    qx = (sx[...] + dx[...] * t - _LO32[0].item()) / _VS32[0].item()
    qy = (sy[...] + dy[...] * t - _LO32[1].item()) / _VS32[1].item()
    qz = (sz[...] + dz[...] * t - _LO32[2].item()) / _VS32[2].item()
    ix = jnp.floor(qx).astype(jnp.int32)
    iy = jnp.floor(qy).astype(jnp.int32)
    iz = jnp.floor(qz).astype(jnp.int32)
    inb = ((ix >= 0) & (ix < _GRID[0]) & (iy >= 0) & (iy < _GRID[1])
           & (iz >= 0) & (iz < _GRID[2]))
    lin = (ix * _GRID[1] + iy) * _GRID[2] + iz
    lin_s = jnp.where(inb, lin, _SENT_LIN)
    tlt = (t < rl[...]).astype(jnp.int32)
    out_ref[...] = (lin_s << 1) | tlt
    smax_ref[...] = jnp.sum(tlt, axis=1, keepdims=True)


def _make_codes(sx, sy, sz, dx, dy, dz, rlen, ts):
    per_ray = pl.BlockSpec((256, 1), lambda i: (i, 0))
    return pl.pallas_call(
        _codes_body,
        grid=(64,),
        in_specs=[per_ray] * 7 + [pl.BlockSpec((1, 256), lambda i: (0, 0))],
        out_specs=(pl.BlockSpec((256, 256), lambda i: (i, 0)), per_ray),
        out_shape=(jax.ShapeDtypeStruct((_N, _N_STEPS), jnp.int32),
                   jax.ShapeDtypeStruct((_N, 1), jnp.int32)),
    )(sx, sy, sz, dx, dy, dz, rlen, ts)


def _sc_body(codes_hbm, smax_hbm, bm_hbm, first_hbm, ncnt_hbm, grid_hbm,
             bm_v, code_a, code_b, idx_v, val_v, zbuf, first_v, ncnt_v,
             smax_v, grid_s, sem_a, sem_b, scat_sem):
    cc = lax.axis_index("c")
    s = lax.axis_index("s")
    lanes = lax.iota(jnp.int32, 16)
    l256 = lanes * _N_STEPS

    def _z(i, _):
        zbuf[pl.ds(i * 16, 16)] = jnp.zeros((16,), jnp.int32)
        return _
    lax.fori_loop(0, 256, _z, None)

    def _zg(k, _):
        pltpu.sync_copy(zbuf, grid_s.at[pl.ds(s * 32768 + k * 4096, 4096)])
        return _
    lax.fori_loop(0, 8, _zg, None)

    pltpu.sync_copy(bm_hbm, bm_v)
    pltpu.sync_copy(smax_hbm.at[pl.ds(s * _RAYS_T, _RAYS_T)], smax_v)
    plsc.subcore_barrier()

    def occ_at(code_v_, sstep):
        code = plsc.load_gather(code_v_, [l256 + sstep])
        c1 = code >> 1
        occw = plsc.load_gather(bm_v, [c1 >> 5])
        occb = lax.shift_right_logical(occw, c1 & 31) & 1
        return code, c1, occb

    def code_slice(ci):
        base = (s * _RAYS_T + ci * _CH_RAYS) * _N_STEPS
        return codes_hbm.at[pl.ds(base, _CH_WORDS)]

    def chunk_body(ci, code_v, code_sem):
        smax16 = smax_v[pl.ds(ci * _CH_RAYS, 16)]
        ka = (jnp.max(smax16) + 7) >> 3

        nf0 = jnp.ones((16,), jnp.int32)
        acc0 = jnp.zeros((16,), jnp.int32)

        def blockA(b, carry):
            nf, first, acc, woff = carry
            cand = []
            for k in range(8):
                code, c1, occb = occ_at(code_v, b * 8 + k)
                nf = nf & (1 - occb)
                first = first + nf
                bb = occb & code & 1
                acc = acc + bb
                sb = (bb == 1) & ((c1 >> 20) == cc)
                u = c1 & (_HALF - 1)
                sval = jnp.where((u >> 19) == 0, 1, 1 << 16)
                cand.append((u & (_QUART - 1), sval, sb,
                             plsc.all_reduce_population_count(sb)[0]))
            for iu, sval, sb, cnt in cand:
                plsc.store_compressed(idx_v.at[pl.ds(woff, 16)], iu, mask=sb)
                plsc.store_compressed(val_v.at[pl.ds(woff, 16)], sval,
                                      mask=sb)
                woff = woff + cnt
            return nf, first, acc, woff

        nf, first, acc, woff = lax.fori_loop(
            0, ka, blockA, (nf0, acc0, acc0, jnp.int32(0)))

        def condB(st):
            v, nf, first = st
            return (v < _N_STEPS) & (jnp.max(nf) > 0)

        def bodyB(st):
            v, nf, first = st
            for k in range(8):
                _, _, occb = occ_at(code_v, v + k)
                nf = nf & (1 - occb)
                first = first + nf
            return v + 8, nf, first

        first = lax.while_loop(condB, bodyB, (ka * 8, nf, first))[2]

        first_v[pl.ds(ci * _CH_RAYS, 16)] = first
        ncnt_v[pl.ds(ci * _CH_RAYS, 16)] = acc

        for k in range(8):
            val_v[pl.ds(woff + k * 16, 16)] = jnp.zeros((16,), jnp.int32)
        nr = (woff + 127) >> 7

        def flush(j, _):
            pltpu.sync_copy(val_v.at[pl.ds(j * 128, 128)],
                            grid_s.at[idx_v.at[pl.ds(j * 128, 128)]],
                            add=True)
            return _
        lax.fori_loop(0, nr, flush, None)

    def _zi(j, _):
        for k in range(8):
            idx_v[pl.ds(j * 128 + k * 16, 16)] = jnp.zeros((16,), jnp.int32)
        return _
    lax.fori_loop(0, 32, _zi, None)

    pltpu.async_copy(code_slice(0), code_a, sem_a)

    def pair_body(i, _):
        ci0 = i * 2
        pltpu.make_async_copy(code_slice(ci0), code_a, sem_a).wait()
        pltpu.async_copy(code_slice(ci0 + 1), code_b, sem_b)
        chunk_body(ci0, code_a, sem_a)

        pltpu.make_async_copy(code_slice(ci0 + 1), code_b, sem_b).wait()

        @pl.when(i < _RAYS_T // _CH_RAYS // 2 - 1)
        def _():
            pltpu.async_copy(code_slice(ci0 + 2), code_a, sem_a)
        chunk_body(ci0 + 1, code_b, sem_b)
        return _

    lax.fori_loop(0, _RAYS_T // _CH_RAYS // 2, pair_body, None)

    @pl.when(cc == 0)
    def _():
        pltpu.sync_copy(first_v, first_hbm.at[pl.ds(s * _RAYS_T, _RAYS_T)])
        pltpu.sync_copy(ncnt_v, ncnt_hbm.at[pl.ds(s * _RAYS_T, _RAYS_T)])

    plsc.subcore_barrier()
    pltpu.sync_copy(grid_s.at[pl.ds(s * 32768, 32768)],
                    grid_hbm.at[cc, pl.ds(s * 32768, 32768)])


def _sc_march(codes, smax, bm):
    mesh = plsc.VectorSubcoreMesh(core_axis_name="c", subcore_axis_name="s",
                                  num_cores=2, num_subcores=16)
    fn = pl.kernel(
        _sc_body,
        out_type=(jax.ShapeDtypeStruct((_N,), jnp.int32),
                  jax.ShapeDtypeStruct((_N,), jnp.int32),
                  jax.ShapeDtypeStruct((2, _QUART), jnp.int32)),
        mesh=mesh,
        compiler_params=pltpu.CompilerParams(needs_layout_passes=False),
        scratch_types=[
            pltpu.VMEM((_BM_PAD,), jnp.int32),
            pltpu.VMEM((_CH_WORDS,), jnp.int32),
            pltpu.VMEM((_CH_WORDS,), jnp.int32),
            pltpu.VMEM((4224,), jnp.int32),
            pltpu.VMEM((4224,), jnp.int32),
            pltpu.VMEM((4096,), jnp.int32),
            pltpu.VMEM((_RAYS_T,), jnp.int32),
            pltpu.VMEM((_RAYS_T,), jnp.int32),
            pltpu.VMEM((_RAYS_T,), jnp.int32),
            pltpu.VMEM_SHARED((_QUART,), jnp.int32),
            pltpu.SemaphoreType.DMA,
            pltpu.SemaphoreType.DMA,
            pltpu.SemaphoreType.DMA,
        ],
    )
    return fn(codes.reshape(_N * _N_STEPS), smax.reshape(_N), bm)


def _combine_body(g_ref, l0_ref, h0_ref, l1_ref, h1_ref):
    g = g_ref[...]
    mask = jnp.int32(0xFFFF)
    l0_ref[...] = (g[0] & mask).astype(jnp.float32)
    h0_ref[...] = lax.shift_right_logical(g[0], 16).astype(jnp.float32)
    l1_ref[...] = (g[1] & mask).astype(jnp.float32)
    h1_ref[...] = lax.shift_right_logical(g[1], 16).astype(jnp.float32)


def _combine(grid2):
    g3 = grid2.reshape(2, 4096, 128)
    spec = pl.BlockSpec((256, 128), lambda i: (i, 0))
    sds = jax.ShapeDtypeStruct((4096, 128), jnp.float32)
    parts = pl.pallas_call(
        _combine_body,
        grid=(16,),
        in_specs=[pl.BlockSpec((2, 256, 128), lambda i: (0, i, 0))],
        out_specs=(spec, spec, spec, spec),
        out_shape=(sds, sds, sds, sds),
    )(g3)
    return jnp.concatenate([p.reshape(_QUART) for p in parts])


def _final_body(first_ref, t_box_ref, mask_ref, ld_ref, rend_ref, fin_ref):
    fs = first_ref[...]
    hh = fs < _N_STEPS
    md = jnp.float32(_MAX_DIST)
    th = (fs.astype(jnp.float32) + 0.5) / _N_STEPS * md
    r = jnp.minimum(jnp.where(hh, th, md), md)
    rend_ref[...] = jnp.minimum(r, t_box_ref[...])
    fin_ref[...] = hh & mask_ref[...] & (ld_ref[...] > _MIN_DISTANCE)


def _finalize(first_s, t_box, mask, ld):
    return pl.pallas_call(
        _final_body,
        out_shape=(jax.ShapeDtypeStruct((1, _N), jnp.float32),
                   jax.ShapeDtypeStruct((1, _N), jnp.bool_)),
    )(first_s, t_box, mask, ld)


def _transform(T, pts):
    return jnp.einsum('bij,bnj->bni', T[:, :3, :3], pts) + T[:, None, :3, 3]


def kernel(occupancy, points_lidar, points_mask, ego_from_lidar):
    B, N, _ = points_lidar.shape

    start = _transform(ego_from_lidar, jnp.zeros_like(points_lidar))
    end = _transform(ego_from_lidar, points_lidar)
    ray = end - start
    ray_len = jnp.linalg.norm(ray, axis=-1)
    dirn = ray / jnp.maximum(ray_len[..., None], 1e-8)
    ts = (jnp.arange(_N_STEPS, dtype=jnp.float32) + 0.5) / _N_STEPS * _MAX_DIST

    occ_t = occupancy.reshape(_NWORDS, 32).T
    bm = _pack_bitmask(occ_t)

    col = lambda a: a.reshape(N, 1)
    codes, smax = _make_codes(col(start[0, :, 0]), col(start[0, :, 1]),
                              col(start[0, :, 2]), col(dirn[0, :, 0]),
                              col(dirn[0, :, 1]), col(dirn[0, :, 2]),
                              col(ray_len[0]), ts.reshape(1, _N_STEPS))

    first_s, ncnt, grid2 = _sc_march(codes, smax, bm)

    voxel_trans = _combine(grid2).reshape(B, *_GRID)
    num_transmissions = ncnt.astype(jnp.float32).reshape(B, N)

    safe = jnp.where(jnp.abs(dirn) < 1e-8, 1e-8, dirn)
    t_up = (_UPPER - start) / safe
    t_lo = (_LOWER - start) / safe
    t_exit = jnp.where(dirn > 0, t_up, t_lo)
    t_exit = jnp.where(jnp.abs(dirn) < 1e-8, jnp.full_like(t_exit, 1e9), t_exit)
    t_box = jnp.maximum(jnp.min(t_exit, axis=-1), 0.0)

    lidar_depth = jnp.linalg.norm(points_lidar, axis=-1)
    in_vol = lambda p: jnp.all((p >= _LOWER) & (p <= _UPPER), axis=-1)
    points_in_volume = (in_vol(end) & in_vol(start) & points_mask
                        & (lidar_depth > _MIN_DISTANCE))

    rendered, finite_depth = _finalize(first_s.reshape(B, N), t_box,
                                       points_mask, lidar_depth)

    return (rendered, num_transmissions, lidar_depth, points_in_volume,
            finite_depth, voxel_trans)


# speedup vs baseline: 1.2813x; 1.0000x over previous
import jax
import jax.numpy as jnp
import numpy as np
from jax import lax
from jax.experimental import pallas as pl
from jax.experimental.pallas import tpu as pltpu
from jax.experimental.pallas import tpu_sc as plsc

_GRID = (256, 256, 32)
_FREE_INDEX = 17
_MIN_DISTANCE = 2.5
_N_STEPS = 256
_NVOX = _GRID[0] * _GRID[1] * _GRID[2]
_HALF = _NVOX // 2
_SENT_LIN = _NVOX
_NWORDS = _NVOX // 32
_BM_PAD = _NWORDS + 128

_LO32 = np.array([-51.2, -51.2, -3.0], dtype=np.float32)
_UP32 = np.array([51.2, 51.2, 3.4], dtype=np.float32)
_VS32 = (_UP32 - _LO32) / np.array(_GRID, dtype=np.float32)
_MAX_DIST = float(np.sqrt(np.sum(np.square(_UP32 - _LO32), dtype=np.float32)))

_LOWER = jnp.asarray(_LO32)
_UPPER = jnp.asarray(_UP32)

_N = 16384
_QUART = _NVOX // 4
_RAYS_T = _N // 16
_CH_RAYS = 16
_CH_WORDS = _CH_RAYS * _N_STEPS


def _pack_body(occ_ref, out_ref):
    bits = (occ_ref[...] != _FREE_INDEX).astype(jnp.int32)
    z = lax.broadcasted_iota(jnp.int32, bits.shape, 0)
    out_ref[...] = jnp.sum(bits << z, axis=0, keepdims=True)[None]


def _pack_bitmask(occ_t):
    words = pl.pallas_call(
        _pack_body,
        grid=(64,),
        in_specs=[pl.BlockSpec((32, 1024), lambda i: (0, i))],
        out_specs=pl.BlockSpec((1, 1, 1024), lambda i: (i, 0, 0)),
        out_shape=jax.ShapeDtypeStruct((64, 1, 1024), jnp.int32),
    )(occ_t)
    return jnp.concatenate(
        [words.reshape(_NWORDS), jnp.zeros((_BM_PAD - _NWORDS,), jnp.int32)])


def _codes_body(sx, sy, sz, dx, dy, dz, rl, ts_ref, out_ref, smax_ref):
    t = ts_ref[...]
    qx = (sx[...] + dx[...] * t - _LO32[0].item()) / _VS32[0].item()
    qy = (sy[...] + dy[...] * t - _LO32[1].item()) / _VS32[1].item()
    qz = (sz[...] + dz[...] * t - _LO32[2].item()) / _VS32[2].item()
    ix = jnp.floor(qx).astype(jnp.int32)
    iy = jnp.floor(qy).astype(jnp.int32)
    iz = jnp.floor(qz).astype(jnp.int32)
    inb = ((ix >= 0) & (ix < _GRID[0]) & (iy >= 0) & (iy < _GRID[1])
           & (iz >= 0) & (iz < _GRID[2]))
    lin = (ix * _GRID[1] + iy) * _GRID[2] + iz
    lin_s = jnp.where(inb, lin, _SENT_LIN)
    tlt = (t < rl[...]).astype(jnp.int32)
    out_ref[...] = (lin_s << 1) | tlt
    smax_ref[...] = jnp.sum(tlt, axis=1, keepdims=True)


def _make_codes(sx, sy, sz, dx, dy, dz, rlen, ts):
    per_ray = pl.BlockSpec((256, 1), lambda i: (i, 0))
    return pl.pallas_call(
        _codes_body,
        grid=(64,),
        in_specs=[per_ray] * 7 + [pl.BlockSpec((1, 256), lambda i: (0, 0))],
        out_specs=(pl.BlockSpec((256, 256), lambda i: (i, 0)), per_ray),
        out_shape=(jax.ShapeDtypeStruct((_N, _N_STEPS), jnp.int32),
                   jax.ShapeDtypeStruct((_N, 1), jnp.int32)),
    )(sx, sy, sz, dx, dy, dz, rlen, ts)


def _sc_body(codes_hbm, smax_hbm, bm_hbm, first_hbm, ncnt_hbm, grid_hbm,
             bm_v, code_a, code_b, idx_v, val_v, zbuf, first_v, ncnt_v,
             smax_v, grid_s, sem_a, sem_b):
    cc = lax.axis_index("c")
    s = lax.axis_index("s")
    lanes = lax.iota(jnp.int32, 16)
    l256 = lanes * _N_STEPS

    def _z(i, _):
        zbuf[pl.ds(i * 16, 16)] = jnp.zeros((16,), jnp.int32)
        return _
    lax.fori_loop(0, 256, _z, None)

    def _zg(k, _):
        pltpu.sync_copy(zbuf, grid_s.at[pl.ds(s * 32768 + k * 4096, 4096)])
        return _
    lax.fori_loop(0, 8, _zg, None)

    pltpu.sync_copy(bm_hbm, bm_v)
    pltpu.sync_copy(smax_hbm.at[pl.ds(s * _RAYS_T, _RAYS_T)], smax_v)
    plsc.subcore_barrier()

    def occ_at(code_v_, sstep):
        code = plsc.load_gather(code_v_, [l256 + sstep])
        c1 = code >> 1
        occw = plsc.load_gather(bm_v, [c1 >> 5])
        occb = lax.shift_right_logical(occw, c1 & 31) & 1
        return code, c1, occb

    def code_slice(ci):
        base = (s * _RAYS_T + ci * _CH_RAYS) * _N_STEPS
        return codes_hbm.at[pl.ds(base, _CH_WORDS)]

    def chunk_body(ci, code_v):
        smax16 = smax_v[pl.ds(ci * _CH_RAYS, 16)]
        ka = (jnp.max(smax16) + 7) >> 3

        nf0 = jnp.ones((16,), jnp.int32)
        acc0 = jnp.zeros((16,), jnp.int32)

        def blockA(b, carry):
            nf, first, acc, woff = carry
            cand = []
            for k in range(8):
                code, c1, occb = occ_at(code_v, b * 8 + k)
                nf = nf & (1 - occb)
                first = first + nf
                bb = occb & code & 1
                acc = acc + bb
                sb = (bb == 1) & ((c1 >> 20) == cc)
                u = c1 & (_HALF - 1)
                sval = jnp.where((u >> 19) == 0, 1, 1 << 16)
                cand.append((u & (_QUART - 1), sval, sb,
                             plsc.all_reduce_population_count(sb)[0]))
            for iu, sval, sb, cnt in cand:
                plsc.store_compressed(idx_v.at[pl.ds(woff, 16)], iu, mask=sb)
                plsc.store_compressed(val_v.at[pl.ds(woff, 16)], sval,
                                      mask=sb)
                woff = woff + cnt
            return nf, first, acc, woff

        nf, first, acc, woff = lax.fori_loop(
            0, ka, blockA, (nf0, acc0, acc0, jnp.int32(0)))

        def condB(st):
            v, nf, first = st
            return (v < _N_STEPS) & (jnp.max(nf) > 0)

        def bodyB(st):
            v, nf, first = st
            for k in range(8):
                _, _, occb = occ_at(code_v, v + k)
                nf = nf & (1 - occb)
                first = first + nf
            return v + 8, nf, first

        first = lax.while_loop(condB, bodyB, (ka * 8, nf, first))[2]

        first_v[pl.ds(ci * _CH_RAYS, 16)] = first
        ncnt_v[pl.ds(ci * _CH_RAYS, 16)] = acc

        for k in range(8):
            val_v[pl.ds(woff + k * 16, 16)] = jnp.zeros((16,), jnp.int32)
        nr = (woff + 127) >> 7

        def flush(j, _):
            pltpu.sync_copy(val_v.at[pl.ds(j * 128, 128)],
                            grid_s.at[idx_v.at[pl.ds(j * 128, 128)]],
                            add=True)
            return _
        lax.fori_loop(0, nr, flush, None)

    def _zi(j, _):
        for k in range(8):
            idx_v[pl.ds(j * 128 + k * 16, 16)] = jnp.zeros((16,), jnp.int32)
        return _
    lax.fori_loop(0, 32, _zi, None)

    pltpu.async_copy(code_slice(0), code_a, sem_a)

    def pair_body(i, _):
        ci0 = i * 2
        pltpu.make_async_copy(code_slice(ci0), code_a, sem_a).wait()
        pltpu.async_copy(code_slice(ci0 + 1), code_b, sem_b)
        chunk_body(ci0, code_a)

        pltpu.make_async_copy(code_slice(ci0 + 1), code_b, sem_b).wait()

        @pl.when(i < _RAYS_T // _CH_RAYS // 2 - 1)
        def _():
            pltpu.async_copy(code_slice(ci0 + 2), code_a, sem_a)
        chunk_body(ci0 + 1, code_b)
        return _

    lax.fori_loop(0, _RAYS_T // _CH_RAYS // 2, pair_body, None)

    @pl.when(cc == 0)
    def _():
        pltpu.sync_copy(first_v, first_hbm.at[pl.ds(s * _RAYS_T, _RAYS_T)])
        pltpu.sync_copy(ncnt_v, ncnt_hbm.at[pl.ds(s * _RAYS_T, _RAYS_T)])

    plsc.subcore_barrier()
    pltpu.sync_copy(grid_s.at[pl.ds(s * 32768, 32768)],
                    grid_hbm.at[cc, pl.ds(s * 32768, 32768)])


def _sc_march(codes, smax, bm):
    mesh = plsc.VectorSubcoreMesh(core_axis_name="c", subcore_axis_name="s",
                                  num_cores=2, num_subcores=16)
    fn = pl.kernel(
        _sc_body,
        out_type=(jax.ShapeDtypeStruct((_N,), jnp.int32),
                  jax.ShapeDtypeStruct((_N,), jnp.int32),
                  jax.ShapeDtypeStruct((2, _QUART), jnp.int32)),
        mesh=mesh,
        compiler_params=pltpu.CompilerParams(needs_layout_passes=False),
        scratch_types=[
            pltpu.VMEM((_BM_PAD,), jnp.int32),
            pltpu.VMEM((_CH_WORDS,), jnp.int32),
            pltpu.VMEM((_CH_WORDS,), jnp.int32),
            pltpu.VMEM((4224,), jnp.int32),
            pltpu.VMEM((4224,), jnp.int32),
            pltpu.VMEM((4096,), jnp.int32),
            pltpu.VMEM((_RAYS_T,), jnp.int32),
            pltpu.VMEM((_RAYS_T,), jnp.int32),
            pltpu.VMEM((_RAYS_T,), jnp.int32),
            pltpu.VMEM_SHARED((_QUART,), jnp.int32),
            pltpu.SemaphoreType.DMA,
            pltpu.SemaphoreType.DMA,
        ],
    )
    return fn(codes.reshape(_N * _N_STEPS), smax.reshape(_N), bm)


def _combine_body(g_ref, l0_ref, h0_ref, l1_ref, h1_ref):
    g = g_ref[...]
    mask = jnp.int32(0xFFFF)
    l0_ref[...] = (g[0] & mask).astype(jnp.float32)
    h0_ref[...] = lax.shift_right_logical(g[0], 16).astype(jnp.float32)
    l1_ref[...] = (g[1] & mask).astype(jnp.float32)
    h1_ref[...] = lax.shift_right_logical(g[1], 16).astype(jnp.float32)


def _combine(grid2):
    g3 = grid2.reshape(2, 4096, 128)
    spec = pl.BlockSpec((256, 128), lambda i: (i, 0))
    sds = jax.ShapeDtypeStruct((4096, 128), jnp.float32)
    parts = pl.pallas_call(
        _combine_body,
        grid=(16,),
        in_specs=[pl.BlockSpec((2, 256, 128), lambda i: (0, i, 0))],
        out_specs=(spec, spec, spec, spec),
        out_shape=(sds, sds, sds, sds),
    )(g3)
    return jnp.concatenate([p.reshape(_QUART) for p in parts])


def _final_body(first_ref, t_box_ref, mask_ref, ld_ref, rend_ref, fin_ref):
    fs = first_ref[...]
    hh = fs < _N_STEPS
    md = jnp.float32(_MAX_DIST)
    th = (fs.astype(jnp.float32) + 0.5) / _N_STEPS * md
    r = jnp.minimum(jnp.where(hh, th, md), md)
    rend_ref[...] = jnp.minimum(r, t_box_ref[...])
    fin_ref[...] = hh & mask_ref[...] & (ld_ref[...] > _MIN_DISTANCE)


def _finalize(first_s, t_box, mask, ld):
    return pl.pallas_call(
        _final_body,
        out_shape=(jax.ShapeDtypeStruct((1, _N), jnp.float32),
                   jax.ShapeDtypeStruct((1, _N), jnp.bool_)),
    )(first_s, t_box, mask, ld)


def _transform(T, pts):
    return jnp.einsum('bij,bnj->bni', T[:, :3, :3], pts) + T[:, None, :3, 3]


def kernel(occupancy, points_lidar, points_mask, ego_from_lidar):
    B, N, _ = points_lidar.shape

    start = _transform(ego_from_lidar, jnp.zeros_like(points_lidar))
    end = _transform(ego_from_lidar, points_lidar)
    ray = end - start
    ray_len = jnp.linalg.norm(ray, axis=-1)
    dirn = ray / jnp.maximum(ray_len[..., None], 1e-8)
    ts = (jnp.arange(_N_STEPS, dtype=jnp.float32) + 0.5) / _N_STEPS * _MAX_DIST

    occ_t = occupancy.reshape(_NWORDS, 32).T
    bm = _pack_bitmask(occ_t)

    col = lambda a: a.reshape(N, 1)
    codes, smax = _make_codes(col(start[0, :, 0]), col(start[0, :, 1]),
                              col(start[0, :, 2]), col(dirn[0, :, 0]),
                              col(dirn[0, :, 1]), col(dirn[0, :, 2]),
                              col(ray_len[0]), ts.reshape(1, _N_STEPS))

    first_s, ncnt, grid2 = _sc_march(codes, smax, bm)

    voxel_trans = _combine(grid2).reshape(B, *_GRID)
    num_transmissions = ncnt.astype(jnp.float32).reshape(B, N)

    safe = jnp.where(jnp.abs(dirn) < 1e-8, 1e-8, dirn)
    t_up = (_UPPER - start) / safe
    t_lo = (_LOWER - start) / safe
    t_exit = jnp.where(dirn > 0, t_up, t_lo)
    t_exit = jnp.where(jnp.abs(dirn) < 1e-8, jnp.full_like(t_exit, 1e9), t_exit)
    t_box = jnp.maximum(jnp.min(t_exit, axis=-1), 0.0)

    lidar_depth = jnp.linalg.norm(points_lidar, axis=-1)
    in_vol = lambda p: jnp.all((p >= _LOWER) & (p <= _UPPER), axis=-1)
    points_in_volume = (in_vol(end) & in_vol(start) & points_mask
                        & (lidar_depth > _MIN_DISTANCE))

    rendered, finite_depth = _finalize(first_s.reshape(B, N), t_box,
                                       points_mask, lidar_depth)

    return (rendered, num_transmissions, lidar_depth, points_in_volume,
            finite_depth, voxel_trans)
